# Initial kernel scaffold; baseline (speedup 1.0000x reference)
import jax
import jax.numpy as jnp
from jax.experimental import pallas as pl


def kernel(points_coords, centers_coords, centers_features, points_features, sim_w1, sim_b1, sim_g1, sim_be1, sim_w2, sim_b2, mlp_w, mlp_b, mlp_g, mlp_be):
    raise NotImplementedError("write your pallas kernel here")


# speedup vs baseline: 15.7411x; 15.7411x over previous
import functools

import jax
import jax.numpy as jnp
from jax.experimental import pallas as pl

_K = 8
_TN = 256


def _knn_h_body(pc_ref, cc_ref, pft_ref, ptab_ref, w1aT_ref, w1bT_ref, b1_ref,
                idx_ref, h_ref, hs_ref, hss_ref, *, M, K, TN):
    b = pl.program_id(0)
    t = pl.program_id(1)
    pc = pc_ref[0]
    cc = cc_ref[0]
    d = (jnp.sum(pc * pc, axis=1, keepdims=True)
         + jnp.sum(cc * cc, axis=0, keepdims=True)
         - 2.0 * jnp.dot(pc, cc, preferred_element_type=jnp.float32))
    gt = jnp.dot(ptab_ref[0], w1aT_ref[...],
                 preferred_element_type=jnp.float32)
    hsh = jnp.dot(pft_ref[0], w1bT_ref[...],
                  preferred_element_type=jnp.float32) + b1_ref[...]
    iota = jax.lax.broadcasted_iota(jnp.int32, (TN, M), 1)
    cols = []
    acc_s = jnp.zeros_like(hs_ref)
    acc_ss = jnp.zeros_like(hss_ref)
    for k in range(K):
        dmin = jnp.min(d, axis=1, keepdims=True)
        ik = jnp.min(jnp.where(d == dmin, iota, M), axis=1, keepdims=True)
        mk = iota == ik
        hk = jnp.dot(mk.astype(jnp.float32), gt,
                     preferred_element_type=jnp.float32) + hsh
        h_ref[0, k] = hk
        acc_s = acc_s + jnp.sum(hk, axis=0, keepdims=True)
        acc_ss = acc_ss + jnp.sum(hk * hk, axis=0, keepdims=True)
        cols.append(ik)
        d = jnp.where(mk, jnp.inf, d)
    idx_ref[0] = jnp.concatenate(cols, axis=1)

    @pl.when((b == 0) & (t == 0))
    def _():
        hs_ref[...] = jnp.zeros_like(hs_ref)
        hss_ref[...] = jnp.zeros_like(hss_ref)

    hs_ref[...] += acc_s
    hss_ref[...] += acc_ss


def _interp_mlp_body(h_ref, idx_ref, pft_ref, ctab_ref, sc1_ref, sh1_ref,
                     w2T_ref, b2_ref, w64T_ref, w32T_ref, mb_ref,
                     w_ref, y_ref, ys_ref, yss_ref, *, M, K, TN):
    b = pl.program_id(0)
    t = pl.program_id(1)
    iota = jax.lax.broadcasted_iota(jnp.int32, (TN, M), 1)
    idx = idx_ref[0]
    wc = jnp.zeros((TN, M), jnp.float32)
    wcols = []
    for k in range(K):
        hn = h_ref[0, k] * sc1_ref[...] + sh1_ref[...]
        hr = jnp.maximum(hn, 0.0)
        s = jnp.dot(hr, w2T_ref[...],
                    preferred_element_type=jnp.float32) + b2_ref[...]
        wk = 1.0 / (1.0 + jnp.exp(-s))
        wcols.append(wk)
        mk = (iota == idx[:, k:k + 1]).astype(jnp.float32)
        wc = wc + wk * mk
    wall = jnp.concatenate(wcols, axis=1)
    wsum = jnp.sum(wall, axis=1, keepdims=True)
    w_ref[0] = wall
    interp = (jnp.dot(wc, ctab_ref[0], preferred_element_type=jnp.float32)
              * (1.0 / (wsum + 1e-8)))
    y = (jnp.dot(interp, w64T_ref[...], preferred_element_type=jnp.float32)
         + jnp.dot(pft_ref[0], w32T_ref[...], preferred_element_type=jnp.float32)
         + mb_ref[...])
    y_ref[0] = y

    @pl.when((b == 0) & (t == 0))
    def _():
        ys_ref[...] = jnp.zeros_like(ys_ref)
        yss_ref[...] = jnp.zeros_like(yss_ref)

    ys_ref[...] += jnp.sum(y, axis=0, keepdims=True)
    yss_ref[...] += jnp.sum(y * y, axis=0, keepdims=True)


def _bn_relu_body(y_ref, sc_ref, sh_ref, o_ref):
    o_ref[0] = jnp.maximum(y_ref[0] * sc_ref[...] + sh_ref[...], 0.0)


def kernel(points_coords, centers_coords, centers_features, points_features,
           sim_w1, sim_b1, sim_g1, sim_be1, sim_w2, sim_b2,
           mlp_w, mlp_b, mlp_g, mlp_be):
    B, _, N = points_coords.shape
    M = centers_coords.shape[2]
    Csa = points_features.shape[1]
    Cin = centers_features.shape[1]
    Cout = mlp_w.shape[0]
    eps = 1e-5
    K = _K
    TN = min(_TN, N)
    nt = N // TN

    pct = jnp.transpose(points_coords, (0, 2, 1))
    pft = jnp.transpose(points_features, (0, 2, 1))
    ptab = pft[:, :M, :]
    ctab = jnp.transpose(centers_features, (0, 2, 1))
    w1aT = jnp.transpose(sim_w1[:, :Csa])
    w1bT = jnp.transpose(sim_w1[:, Csa:])
    b1 = sim_b1[None, :]
    w2T = jnp.transpose(sim_w2)
    b2 = sim_b2[None, :]
    w64T = jnp.transpose(mlp_w[:, :Cin])
    w32T = jnp.transpose(mlp_w[:, Cin:])
    mb = mlp_b[None, :]

    grid = (B, nt)
    const = lambda b, t: (0, 0)

    idx, h, hs, hss = pl.pallas_call(
        functools.partial(_knn_h_body, M=M, K=K, TN=TN),
        grid=grid,
        in_specs=[
            pl.BlockSpec((1, TN, 3), lambda b, t: (b, t, 0)),
            pl.BlockSpec((1, 3, M), lambda b, t: (b, 0, 0)),
            pl.BlockSpec((1, TN, Csa), lambda b, t: (b, t, 0)),
            pl.BlockSpec((1, M, Csa), lambda b, t: (b, 0, 0)),
            pl.BlockSpec((Csa, Csa), const),
            pl.BlockSpec((Csa, Csa), const),
            pl.BlockSpec((1, Csa), const),
        ],
        out_specs=[
            pl.BlockSpec((1, TN, K), lambda b, t: (b, t, 0)),
            pl.BlockSpec((1, K, TN, Csa), lambda b, t: (b, 0, t, 0)),
            pl.BlockSpec((1, Csa), const),
            pl.BlockSpec((1, Csa), const),
        ],
        out_shape=[
            jax.ShapeDtypeStruct((B, N, K), jnp.int32),
            jax.ShapeDtypeStruct((B, K, N, Csa), jnp.float32),
            jax.ShapeDtypeStruct((1, Csa), jnp.float32),
            jax.ShapeDtypeStruct((1, Csa), jnp.float32),
        ],
    )(pct, centers_coords, pft, ptab, w1aT, w1bT, b1)

    cnt1 = B * K * N
    mu1 = hs[0] / cnt1
    var1 = hss[0] / cnt1 - mu1 * mu1
    rstd1 = jax.lax.rsqrt(var1 + eps)
    sc1 = (sim_g1 * rstd1)[None, :]
    sh1 = (sim_be1 - mu1 * sim_g1 * rstd1)[None, :]

    w_nk, y_pre, ys, yss = pl.pallas_call(
        functools.partial(_interp_mlp_body, M=M, K=K, TN=TN),
        grid=grid,
        in_specs=[
            pl.BlockSpec((1, K, TN, Csa), lambda b, t: (b, 0, t, 0)),
            pl.BlockSpec((1, TN, K), lambda b, t: (b, t, 0)),
            pl.BlockSpec((1, TN, Csa), lambda b, t: (b, t, 0)),
            pl.BlockSpec((1, M, Cin), lambda b, t: (b, 0, 0)),
            pl.BlockSpec((1, Csa), const),
            pl.BlockSpec((1, Csa), const),
            pl.BlockSpec((Csa, 1), const),
            pl.BlockSpec((1, 1), const),
            pl.BlockSpec((Cin, Cout), const),
            pl.BlockSpec((Csa, Cout), const),
            pl.BlockSpec((1, Cout), const),
        ],
        out_specs=[
            pl.BlockSpec((1, TN, K), lambda b, t: (b, t, 0)),
            pl.BlockSpec((1, TN, Cout), lambda b, t: (b, t, 0)),
            pl.BlockSpec((1, Cout), const),
            pl.BlockSpec((1, Cout), const),
        ],
        out_shape=[
            jax.ShapeDtypeStruct((B, N, K), jnp.float32),
            jax.ShapeDtypeStruct((B, N, Cout), jnp.float32),
            jax.ShapeDtypeStruct((1, Cout), jnp.float32),
            jax.ShapeDtypeStruct((1, Cout), jnp.float32),
        ],
    )(h, idx, pft, ctab, sc1, sh1, w2T, b2, w64T, w32T, mb)

    cnt2 = B * N
    mu2 = ys[0] / cnt2
    var2 = yss[0] / cnt2 - mu2 * mu2
    rstd2 = jax.lax.rsqrt(var2 + eps)
    sc2 = (mlp_g * rstd2)[None, :]
    sh2 = (mlp_be - mu2 * mlp_g * rstd2)[None, :]

    y_nk = pl.pallas_call(
        _bn_relu_body,
        grid=grid,
        in_specs=[
            pl.BlockSpec((1, TN, Cout), lambda b, t: (b, t, 0)),
            pl.BlockSpec((1, Cout), const),
            pl.BlockSpec((1, Cout), const),
        ],
        out_specs=pl.BlockSpec((1, TN, Cout), lambda b, t: (b, t, 0)),
        out_shape=jax.ShapeDtypeStruct((B, N, Cout), jnp.float32),
    )(y_pre, sc2, sh2)

    y = jnp.transpose(y_nk, (0, 2, 1))
    weights = jnp.transpose(w_nk, (0, 2, 1))
    return (y, points_coords, idx, weights)
